# baseline (speedup 1.0000x reference)
import jax
import jax.numpy as jnp
from jax.experimental import pallas as pl
from jax.experimental.pallas import tpu as pltpu

_B, _T, _D = 4, 3072, 1024
_RUN = 32
_GPB = _T // _RUN


def _body(idx_ref, x_ref, o_ref):
    o_ref[...] = x_ref[...]


@jax.jit
def _tc_permute(x, idx):
    grid_spec = pltpu.PrefetchScalarGridSpec(
        num_scalar_prefetch=1,
        grid=(_GPB,),
        in_specs=[
            pl.BlockSpec(
                (_B, _RUN, _D),
                lambda g, idx_ref: (0, idx_ref[g * _RUN] // _RUN, 0),
            )
        ],
        out_specs=pl.BlockSpec((_B, _RUN, _D), lambda g, idx_ref: (0, g, 0)),
    )
    return pl.pallas_call(
        _body,
        grid_spec=grid_spec,
        out_shape=jax.ShapeDtypeStruct((_B, _T, _D), jnp.float32),
    )(idx, x)


def kernel(x, forward_shuffle_idx):
    return _tc_permute(x, forward_shuffle_idx.astype(jnp.int32))
